# baseline (speedup 1.0000x reference)
import functools
import math

import jax
import jax.numpy as jnp
from jax import lax
from jax.experimental import pallas as pl
from jax.experimental.pallas import tpu as pltpu
from jax.experimental.pallas import tpu_sc as plsc

_SMOOTHING = 0.1
_PAD = 0

_NC = 2
_NS = 16
_NW = _NC * _NS

_M = 4
_BR = 8


def _sc_count_body(tgt_hbm, out_hbm, tgt_v, res_v, *, rows_pw):
    wid = lax.axis_index("s") * _NC + lax.axis_index("c")
    base = wid * rows_pw
    pltpu.sync_copy(tgt_hbm.at[pl.ds(base, rows_pw)], tgt_v)
    nv = jnp.zeros((16,), jnp.float32)
    for c in range(rows_pw // 16):
        tv = tgt_v[pl.ds(c * 16, 16)]
        nv = nv + jnp.where(tv != _PAD, 1.0, 0.0)
    res_v[...] = nv
    pltpu.sync_copy(res_v, out_hbm.at[pl.ds(wid * 16, 16)])


def _sc_valid_count(tgt, n):
    rows_pw = n // _NW
    mesh = plsc.VectorSubcoreMesh(core_axis_name="c", subcore_axis_name="s",
                                  num_cores=_NC, num_subcores=_NS)
    kfn = pl.kernel(
        functools.partial(_sc_count_body, rows_pw=rows_pw),
        out_type=jax.ShapeDtypeStruct((_NW * 16,), jnp.float32),
        mesh=mesh,
        scratch_types=[
            pltpu.VMEM((rows_pw,), jnp.int32),
            pltpu.VMEM((16,), jnp.float32),
        ],
    )
    return kfn(tgt)


def _tc_body(tgt_ref, *refs, eps, m):
    x_refs = refs[:m]
    out_ref, acc_ref = refs[m], refs[m + 1]
    k = pl.program_id(0)
    grid = pl.num_programs(0)

    @pl.when(k == 0)
    def _():
        acc_ref[0] = 0.0

    partial = 0.0
    for i, x_ref in enumerate(x_refs):
        br = x_ref.shape[0]
        row0 = (i * grid + k) * br
        tgt = tgt_ref[pl.ds(row0, br), :]
        vf = (tgt != _PAD).astype(jnp.float32)[:, 0]
        xb = x_ref[...]
        rowsum = jnp.sum(xb, axis=1)
        cols = jax.lax.broadcasted_iota(jnp.int32, xb.shape, 1)
        tgtv = jnp.sum(jnp.where(cols == tgt, xb, 0.0), axis=1)
        col0 = xb[:, 0]
        per_row = (-eps) * rowsum + eps * col0 \
            + (eps - (1.0 - _SMOOTHING)) * tgtv
        partial += jnp.sum(vf * per_row)

    acc_ref[0] += partial

    @pl.when(k == pl.num_programs(0) - 1)
    def _():
        out_ref[0, 0] = acc_ref[0]


def _combine_body(tc_ref, nv_ref, out_ref, *, cval):
    out_ref[0, 0] = tc_ref[0, 0] + cval * jnp.sum(nv_ref[...])


def kernel(x, target):
    n, v = x.shape
    eps = _SMOOTHING / (v - 2)
    cval = _SMOOTHING * math.log(eps) + (1.0 - _SMOOTHING) * math.log(1.0 - _SMOOTHING)
    tgt = target.astype(jnp.int32)
    tgt2d = tgt.reshape(n, 1)

    if n % (_M * _BR) == 0:
        m, br = _M, _BR
    else:
        m, br = 1, 8
    grid = n // (m * br)

    def mk_spec(i):
        return pl.BlockSpec((br, v), lambda k, i=i: (i * grid + k, 0))

    tc_out = pl.pallas_call(
        functools.partial(_tc_body, eps=eps, m=m),
        grid=(grid,),
        in_specs=[pl.BlockSpec((n, 1), lambda k: (0, 0))]
        + [mk_spec(i) for i in range(m)],
        out_specs=pl.BlockSpec(memory_space=pltpu.SMEM),
        out_shape=jax.ShapeDtypeStruct((1, 1), jnp.float32),
        scratch_shapes=[pltpu.SMEM((1,), jnp.float32)],
        compiler_params=pltpu.CompilerParams(
            dimension_semantics=("arbitrary",),
        ),
    )(tgt2d, *([x] * m))

    if n % (16 * _NW) == 0:
        nv_out = _sc_valid_count(tgt, n).reshape(_NW, 16)
        out = pl.pallas_call(
            functools.partial(_combine_body, cval=cval),
            in_specs=[
                pl.BlockSpec(memory_space=pltpu.SMEM),
                pl.BlockSpec((_NW, 16), lambda: (0, 0)),
            ],
            out_specs=pl.BlockSpec(memory_space=pltpu.SMEM),
            out_shape=jax.ShapeDtypeStruct((1, 1), jnp.float32),
        )(tc_out, nv_out)
        return out.reshape(())

    nv = jnp.sum((tgt != _PAD).astype(jnp.float32))
    return (tc_out.reshape(()) + cval * nv)
